# baseline (speedup 1.0000x reference)
import jax
import jax.numpy as jnp
from jax import lax
from jax.experimental import pallas as pl
from jax.experimental.pallas import tpu as pltpu

_BS = 100
_K = 10


def _gcn_kernel(x_ref, tin_hbm, tout_hbm, ttr_hbm, w_ref,
                a_ref, c0_ref, c1_ref, c2_ref, b_ref, out_ref,
                tin_v, tout_v, ttr_v, sem0, sem1, sem2):
    cp0 = pltpu.make_async_copy(tin_hbm, tin_v, sem0)
    cp1 = pltpu.make_async_copy(tout_hbm, tout_v, sem1)
    cp2 = pltpu.make_async_copy(ttr_hbm, ttr_v, sem2)
    cp0.start()
    cp1.start()
    cp2.start()

    x = x_ref[...]
    alpha = a_ref[0, 0].astype(jnp.float32)
    c0 = c0_ref[0, 0]
    c1 = c1_ref[0, 0]
    c2 = c2_ref[0, 0]
    b = b_ref[0, 0]

    xc = x - jnp.mean(x, axis=1, keepdims=True)
    g = lax.dot_general(xc, xc, (((1,), (1,)), ((), ())),
                        preferred_element_type=jnp.float32)
    v = lax.dot_general(x, w_ref[...], (((1,), (1,)), ((), ())),
                        preferred_element_type=jnp.float32)

    inv_col = lax.rsqrt(jnp.sum(xc * xc, axis=1, keepdims=True))
    inv_row = inv_col.T
    ri = lax.broadcasted_iota(jnp.int32, (_BS, _BS), 0)
    ci = lax.broadcasted_iota(jnp.int32, (_BS, _BS), 1)
    eye = jnp.where(ri == ci, jnp.float32(1.0), jnp.float32(0.0))

    corr = g * inv_col * inv_row
    dist0 = (1.0 - corr) * (1.0 - eye)
    d2 = dist0 * dist0
    sigma = jnp.mean(dist0)
    inter = jnp.exp(d2 * (jnp.float32(-0.5) / (sigma * sigma)))
    fea = (inter - eye) * alpha + eye

    cp0.wait()
    cp1.wait()
    cp2.wait()
    pheno = c0 * tin_v[...] + c1 * tout_v[...] + c2 * ttr_v[...] + eye
    adj_t = fea * pheno.T

    neg = jnp.float32(-jnp.inf)
    t = jnp.full((1, _BS), jnp.inf, jnp.float32)
    for _ in range(_K):
        lt = adj_t < t
        m = jnp.max(jnp.where(lt, adj_t, neg), axis=0, keepdims=True)
        ge = jnp.sum(jnp.where(lt, 0.0, 1.0), axis=0, keepdims=True)
        t = jnp.where(ge <= jnp.float32(_K - 1), m, t)
    adjm_t = jnp.where(adj_t < t, jnp.float32(0.0), adj_t)

    out = jnp.sum(adjm_t * v, axis=0, keepdims=True) + b
    out_ref[...] = out


def kernel(x, alpha, test_in_graph, test_out_graph, train_out_graph, k, c0, c1, c2, W, b):
    del k
    a2 = jnp.reshape(jnp.asarray(alpha), (1, 1))
    vm = pl.BlockSpec(memory_space=pltpu.VMEM)
    hbm = pl.BlockSpec(memory_space=pl.ANY)
    out = pl.pallas_call(
        _gcn_kernel,
        in_specs=[vm, hbm, hbm, hbm, vm, vm, vm, vm, vm, vm],
        out_shape=jax.ShapeDtypeStruct((1, _BS), jnp.float32),
        scratch_shapes=[
            pltpu.VMEM((_BS, _BS), jnp.float32),
            pltpu.VMEM((_BS, _BS), jnp.float32),
            pltpu.VMEM((_BS, _BS), jnp.float32),
            pltpu.SemaphoreType.DMA,
            pltpu.SemaphoreType.DMA,
            pltpu.SemaphoreType.DMA,
        ],
    )(x, test_in_graph, test_out_graph, train_out_graph, W, a2,
      jnp.reshape(c0, (1, 1)), jnp.reshape(c1, (1, 1)),
      jnp.reshape(c2, (1, 1)), jnp.reshape(b, (1, 1)))
    return out[0]


# speedup vs baseline: 1.0916x; 1.0916x over previous
import jax
import jax.numpy as jnp
from jax import lax
from jax.experimental import pallas as pl
from jax.experimental.pallas import tpu as pltpu

_BS = 100
_K = 10


def _gcn_kernel(x_ref, tin_ref, tout_ref, ttr_ref, w_ref,
                a_ref, c0_ref, c1_ref, c2_ref, b_ref, out_ref):
    x = x_ref[...]
    alpha = a_ref[0, 0].astype(jnp.float32)
    c0 = c0_ref[0, 0]
    c1 = c1_ref[0, 0]
    c2 = c2_ref[0, 0]
    b = b_ref[0, 0]

    xc = x - jnp.mean(x, axis=1, keepdims=True)
    g = lax.dot_general(xc, xc, (((1,), (1,)), ((), ())),
                        preferred_element_type=jnp.float32)
    v = lax.dot_general(x, w_ref[...], (((1,), (1,)), ((), ())),
                        preferred_element_type=jnp.float32)

    inv_col = lax.rsqrt(jnp.sum(xc * xc, axis=1, keepdims=True))
    inv_row = inv_col.T
    ri = lax.broadcasted_iota(jnp.int32, (_BS, _BS), 0)
    ci = lax.broadcasted_iota(jnp.int32, (_BS, _BS), 1)
    eye = jnp.where(ri == ci, jnp.float32(1.0), jnp.float32(0.0))
    pheno = c0 * tin_ref[...] + c1 * tout_ref[...] + c2 * ttr_ref[...] + eye
    pheno_t = pheno.T

    corr = g * inv_col * inv_row
    dist0 = (1.0 - corr) * (1.0 - eye)
    sigma = jnp.mean(dist0)
    inter = jnp.exp(-(dist0 * dist0) / (2.0 * sigma * sigma))
    fea = (inter - eye) * alpha + eye

    adj_t = fea * pheno_t

    neg = jnp.float32(-jnp.inf)
    t = jnp.full((1, _BS), jnp.inf, jnp.float32)
    for _ in range(_K):
        lt = adj_t < t
        m = jnp.max(jnp.where(lt, adj_t, neg), axis=0, keepdims=True)
        ge = jnp.sum(jnp.where(lt, 0.0, 1.0), axis=0, keepdims=True)
        t = jnp.where(ge <= jnp.float32(_K - 1), m, t)
    adjm_t = jnp.where(adj_t < t, jnp.float32(0.0), adj_t)

    out = jnp.sum(adjm_t * v, axis=0, keepdims=True) + b
    out_ref[...] = out


def kernel(x, alpha, test_in_graph, test_out_graph, train_out_graph, k, c0, c1, c2, W, b):
    del k
    a2 = jnp.reshape(jnp.asarray(alpha), (1, 1))
    vm = pl.BlockSpec(memory_space=pltpu.VMEM)
    sm = pl.BlockSpec(memory_space=pltpu.SMEM)
    out = pl.pallas_call(
        _gcn_kernel,
        in_specs=[vm, vm, vm, vm, vm, sm, sm, sm, sm, sm],
        out_shape=jax.ShapeDtypeStruct((1, _BS), jnp.float32),
    )(x, test_in_graph, test_out_graph, train_out_graph, W, a2,
      jnp.reshape(c0, (1, 1)), jnp.reshape(c1, (1, 1)),
      jnp.reshape(c2, (1, 1)), jnp.reshape(b, (1, 1)))
    return out[0]
